# baseline (device time: 47960 ns/iter reference)
import jax
import jax.numpy as jnp
from jax import lax
from jax.experimental import pallas as pl
from jax.experimental.pallas import tpu as pltpu

N_DEV = 8
B = 2
SQ = 512
D_MODEL = 768
DH = 64
HQ_LOC = 8
D_LOC = HQ_LOC * DH
CHUNK = SQ // N_DEV


def kernel(x, Wq, K_ext, V_ext, Wo):
    my = lax.axis_index("i")
    wq_loc = lax.dynamic_slice_in_dim(Wq, my * D_LOC, D_LOC, axis=1)
    wo_bf16 = Wo.astype(jnp.bfloat16)
    k_t = jnp.transpose(K_ext, (0, 2, 1, 3))
    v_t = jnp.transpose(V_ext, (0, 2, 1, 3))

    def body(x_ref, wq_ref, k_ref, v_ref, wo_ref, out_ref,
             ctx_ref, g_ref, stage_ref,
             s1_sems, r1_sems, s2_sems, r2_sems):
        p = lax.axis_index("i")

        r = lax.broadcasted_iota(jnp.int32, (SQ, SQ), 0)
        c = lax.broadcasted_iota(jnp.int32, (SQ, SQ), 1)
        bias = jnp.where(((r // DH) % 4) == ((c // DH) % 4), 0.0, -30.0)

        wq = (wq_ref[:] * 0.125).astype(jnp.bfloat16)
        qb = [lax.dot(x_ref[b].astype(jnp.bfloat16), wq,
                      preferred_element_type=jnp.float32).astype(jnp.bfloat16)
              for b in range(B)]

        for qt in range(4):
            rows = slice(128 * qt, 128 * (qt + 1))
            for b in range(B):
                for h in range(HQ_LOC):
                    hc = slice(h * DH, (h + 1) * DH)
                    kh = k_ref[b, h].astype(jnp.bfloat16)
                    s = lax.dot_general(
                        qb[b][rows, hc], kh, (((1,), (1,)), ((), ())),
                        preferred_element_type=jnp.float32) + bias[rows]
                    w = jnp.exp(s).astype(jnp.bfloat16)
                    denom = jnp.sum(w.astype(jnp.float32),
                                    axis=1, keepdims=True)
                    vh = v_ref[b, h].astype(jnp.bfloat16)
                    wv = lax.dot(w, vh, preferred_element_type=jnp.float32)
                    ctx_ref[b, rows, hc] = (wv / denom).astype(jnp.bfloat16)
            for q in range(2 * qt, 2 * qt + 2):
                @pl.when(q != p)
                def _():
                    rdma = pltpu.make_async_remote_copy(
                        src_ref=ctx_ref.at[:, q * CHUNK:(q + 1) * CHUNK, :],
                        dst_ref=stage_ref.at[p],
                        send_sem=s1_sems.at[q], recv_sem=r1_sems.at[p],
                        device_id=(q,), device_id_type=pl.DeviceIdType.MESH,
                    )
                    rdma.start()

        stage_ref[p] = ctx_ref[:, pl.ds(p * CHUNK, CHUNK), :]
        res = None
        for d in range(N_DEV):
            @pl.when(d != p)
            def _():
                pltpu.make_async_remote_copy(
                    src_ref=stage_ref.at[d], dst_ref=stage_ref.at[d],
                    send_sem=s1_sems.at[d], recv_sem=r1_sems.at[d],
                    device_id=(d,), device_id_type=pl.DeviceIdType.MESH,
                ).wait_recv()
            lhs = stage_ref[d].reshape(B * CHUNK, D_LOC)
            part = lax.dot(lhs, wo_ref[d * D_LOC:(d + 1) * D_LOC, :],
                           preferred_element_type=jnp.float32)
            res = part if res is None else res + part
        g_ref[:, pl.ds(p * CHUNK, CHUNK), :] = res.reshape(
            B, CHUNK, D_MODEL).astype(jnp.bfloat16)

        for q in range(N_DEV):
            @pl.when(q != p)
            def _():
                pltpu.make_async_remote_copy(
                    src_ref=g_ref.at[:, pl.ds(p * CHUNK, CHUNK), :],
                    dst_ref=g_ref.at[:, pl.ds(p * CHUNK, CHUNK), :],
                    send_sem=s2_sems.at[q], recv_sem=r2_sems.at[p],
                    device_id=(q,), device_id_type=pl.DeviceIdType.MESH,
                ).start()
        for d in range(N_DEV):
            @pl.when(d != p)
            def _():
                pltpu.make_async_remote_copy(
                    src_ref=g_ref.at[:, d * CHUNK:(d + 1) * CHUNK, :],
                    dst_ref=g_ref.at[:, d * CHUNK:(d + 1) * CHUNK, :],
                    send_sem=s2_sems.at[d], recv_sem=r2_sems.at[d],
                    device_id=(d,), device_id_type=pl.DeviceIdType.MESH,
                ).wait_recv()

        out_ref[:, :, :] = g_ref[:, :, :].astype(jnp.float32)

        for q in range(N_DEV):
            @pl.when(q != p)
            def _():
                pltpu.make_async_remote_copy(
                    src_ref=ctx_ref.at[:, q * CHUNK:(q + 1) * CHUNK, :],
                    dst_ref=stage_ref.at[p],
                    send_sem=s1_sems.at[q], recv_sem=r1_sems.at[p],
                    device_id=(q,), device_id_type=pl.DeviceIdType.MESH,
                ).wait_send()
                pltpu.make_async_remote_copy(
                    src_ref=g_ref.at[:, pl.ds(p * CHUNK, CHUNK), :],
                    dst_ref=g_ref.at[:, pl.ds(p * CHUNK, CHUNK), :],
                    send_sem=s2_sems.at[q], recv_sem=r2_sems.at[p],
                    device_id=(q,), device_id_type=pl.DeviceIdType.MESH,
                ).wait_send()

    return pl.pallas_call(
        body,
        out_shape=jax.ShapeDtypeStruct((B, SQ, D_MODEL), jnp.float32),
        in_specs=[pl.BlockSpec(memory_space=pltpu.VMEM)] * 5,
        out_specs=pl.BlockSpec(memory_space=pltpu.VMEM),
        scratch_shapes=[
            pltpu.VMEM((B, SQ, D_LOC), jnp.bfloat16),
            pltpu.VMEM((B, SQ, D_MODEL), jnp.bfloat16),
            pltpu.VMEM((N_DEV, B, CHUNK, D_LOC), jnp.bfloat16),
            pltpu.SemaphoreType.DMA((N_DEV,)),
            pltpu.SemaphoreType.DMA((N_DEV,)),
            pltpu.SemaphoreType.DMA((N_DEV,)),
            pltpu.SemaphoreType.DMA((N_DEV,)),
        ],
    )(x, wq_loc, k_t, v_t, wo_bf16)


# device time: 43844 ns/iter; 1.0939x vs baseline; 1.0939x over previous
import jax
import jax.numpy as jnp
from jax import lax
from jax.experimental import pallas as pl
from jax.experimental.pallas import tpu as pltpu

N_DEV = 8
B = 2
SQ = 512
D_MODEL = 768
DH = 64
HQ_LOC = 8
D_LOC = HQ_LOC * DH
CHUNK = SQ // N_DEV


def kernel(x, Wq, K_ext, V_ext, Wo):
    my = lax.axis_index("i")
    wq_loc = lax.dynamic_slice_in_dim(Wq, my * D_LOC, D_LOC, axis=1)
    wo_loc = lax.dynamic_slice_in_dim(Wo, my * D_LOC, D_LOC, axis=0)
    k_t = jnp.transpose(K_ext, (0, 2, 1, 3))
    v_t = jnp.transpose(V_ext, (0, 2, 1, 3))

    def body(x_ref, wq_ref, k_ref, v_ref, wo_ref, out_ref,
             acc_ref, ctx_ref, stage_ref,
             s1_sems, r1_sems, s2_sems, r2_sems):
        p = lax.axis_index("i")

        r = lax.broadcasted_iota(jnp.int32, (SQ, SQ), 0)
        c = lax.broadcasted_iota(jnp.int32, (SQ, SQ), 1)
        bias = jnp.where(((r // DH) % 4) == ((c // DH) % 4), 0.0, -30.0)

        wq = (wq_ref[:] * 0.125).astype(jnp.bfloat16)
        wo = wo_ref[:].astype(jnp.bfloat16)
        qb = [lax.dot(x_ref[b].astype(jnp.bfloat16), wq,
                      preferred_element_type=jnp.float32).astype(jnp.bfloat16)
              for b in range(B)]

        for half in range(2):
            rows = slice(256 * half, 256 * (half + 1))
            for b in range(B):
                for h in range(HQ_LOC):
                    hc = slice(h * DH, (h + 1) * DH)
                    kh = k_ref[b, h].astype(jnp.bfloat16)
                    s = lax.dot_general(
                        qb[b][rows, hc], kh, (((1,), (1,)), ((), ())),
                        preferred_element_type=jnp.float32) + bias[rows]
                    w = jnp.exp(s).astype(jnp.bfloat16)
                    denom = jnp.sum(w.astype(jnp.float32),
                                    axis=1, keepdims=True)
                    vh = v_ref[b, h].astype(jnp.bfloat16)
                    wv = lax.dot(w, vh, preferred_element_type=jnp.float32)
                    ctx_ref[b, rows, hc] = (wv / denom).astype(jnp.bfloat16)
                acc_ref[b, rows, :] = lax.dot(
                    ctx_ref[b, rows, :], wo,
                    preferred_element_type=jnp.float32).astype(jnp.bfloat16)
            for q in range(4 * half, 4 * half + 4):
                @pl.when(q != p)
                def _():
                    rdma = pltpu.make_async_remote_copy(
                        src_ref=acc_ref.at[:, q * CHUNK:(q + 1) * CHUNK, :],
                        dst_ref=stage_ref.at[p],
                        send_sem=s1_sems.at[q], recv_sem=r1_sems.at[p],
                        device_id=(q,), device_id_type=pl.DeviceIdType.MESH,
                    )
                    rdma.start()

        stage_ref[p] = acc_ref[:, pl.ds(p * CHUNK, CHUNK), :]
        for d in range(N_DEV):
            @pl.when(d != p)
            def _():
                pltpu.make_async_remote_copy(
                    src_ref=stage_ref.at[d], dst_ref=stage_ref.at[d],
                    send_sem=s1_sems.at[d], recv_sem=r1_sems.at[d],
                    device_id=(d,), device_id_type=pl.DeviceIdType.MESH,
                ).wait_recv()

        red = jnp.sum(stage_ref[:, :, :, :].astype(jnp.float32), axis=0)
        out_ref[:, pl.ds(p * CHUNK, CHUNK), :] = red.astype(jnp.bfloat16)

        for q in range(N_DEV):
            @pl.when(q != p)
            def _():
                pltpu.make_async_remote_copy(
                    src_ref=out_ref.at[:, pl.ds(p * CHUNK, CHUNK), :],
                    dst_ref=out_ref.at[:, pl.ds(p * CHUNK, CHUNK), :],
                    send_sem=s2_sems.at[q], recv_sem=r2_sems.at[p],
                    device_id=(q,), device_id_type=pl.DeviceIdType.MESH,
                ).start()
        for d in range(N_DEV):
            @pl.when(d != p)
            def _():
                pltpu.make_async_remote_copy(
                    src_ref=out_ref.at[:, d * CHUNK:(d + 1) * CHUNK, :],
                    dst_ref=out_ref.at[:, d * CHUNK:(d + 1) * CHUNK, :],
                    send_sem=s2_sems.at[d], recv_sem=r2_sems.at[d],
                    device_id=(d,), device_id_type=pl.DeviceIdType.MESH,
                ).wait_recv()

        for q in range(N_DEV):
            @pl.when(q != p)
            def _():
                pltpu.make_async_remote_copy(
                    src_ref=acc_ref.at[:, q * CHUNK:(q + 1) * CHUNK, :],
                    dst_ref=stage_ref.at[p],
                    send_sem=s1_sems.at[q], recv_sem=r1_sems.at[p],
                    device_id=(q,), device_id_type=pl.DeviceIdType.MESH,
                ).wait_send()
                pltpu.make_async_remote_copy(
                    src_ref=out_ref.at[:, pl.ds(p * CHUNK, CHUNK), :],
                    dst_ref=out_ref.at[:, pl.ds(p * CHUNK, CHUNK), :],
                    send_sem=s2_sems.at[q], recv_sem=r2_sems.at[p],
                    device_id=(q,), device_id_type=pl.DeviceIdType.MESH,
                ).wait_send()

    return pl.pallas_call(
        body,
        out_shape=jax.ShapeDtypeStruct((B, SQ, D_MODEL), jnp.bfloat16),
        in_specs=[pl.BlockSpec(memory_space=pltpu.VMEM)] * 5,
        out_specs=pl.BlockSpec(memory_space=pltpu.VMEM),
        scratch_shapes=[
            pltpu.VMEM((B, SQ, D_MODEL), jnp.bfloat16),
            pltpu.VMEM((B, SQ, D_LOC), jnp.bfloat16),
            pltpu.VMEM((N_DEV, B, CHUNK, D_MODEL), jnp.bfloat16),
            pltpu.SemaphoreType.DMA((N_DEV,)),
            pltpu.SemaphoreType.DMA((N_DEV,)),
            pltpu.SemaphoreType.DMA((N_DEV,)),
            pltpu.SemaphoreType.DMA((N_DEV,)),
        ],
    )(x, wq_loc, k_t, v_t, wo_loc)
